# Initial kernel scaffold; baseline (speedup 1.0000x reference)
import jax
import jax.numpy as jnp
from jax.experimental import pallas as pl


def kernel(input_ids, position_ids, token_type_ids, word_embeddings, position_embeddings, token_type_embeddings, ln_gamma, ln_beta):
    raise NotImplementedError("write your pallas kernel here")


# speedup vs baseline: 1.4811x; 1.4811x over previous
import functools

import jax
import jax.numpy as jnp
from jax import lax
from jax.experimental import pallas as pl
from jax.experimental.pallas import tpu as pltpu
from jax.experimental.pallas import tpu_sc as plsc

H = 128
LANES = 16
NJ = H // LANES
EPS = 1e-12


def _rsqrt_scalar(x):
    i = lax.bitcast_convert_type(x, jnp.int32)
    i = jnp.int32(0x5F3759DF) - (i >> 1)
    y = lax.bitcast_convert_type(i, jnp.float32)
    for _ in range(3):
        y = y * (1.5 - 0.5 * x * y * y)
    return y


def _make_sc_call(n_rows, v, p):
    info = plsc.get_sparse_core_info()
    nw = info.num_cores * info.num_subcores
    rows_per_w = n_rows // nw
    chunk = 128
    n_chunks = rows_per_w // chunk
    mesh = plsc.VectorSubcoreMesh(core_axis_name="c", subcore_axis_name="s")

    @functools.partial(
        pl.kernel,
        out_type=jax.ShapeDtypeStruct((n_rows, H), jnp.float32),
        mesh=mesh,
        scratch_types=[
            pltpu.VMEM((chunk,), jnp.int32),
            pltpu.VMEM((chunk,), jnp.int32),
            pltpu.VMEM((chunk,), jnp.int32),
            pltpu.VMEM((chunk, H), jnp.float32),
            pltpu.VMEM((p * H,), jnp.float32),
            pltpu.VMEM((2 * H,), jnp.float32),
            pltpu.VMEM((H,), jnp.float32),
            pltpu.VMEM((H,), jnp.float32),
            pltpu.SemaphoreType.DMA,
        ],
        compiler_params=pltpu.CompilerParams(needs_layout_passes=False),
    )
    def sc_call(ids_h, pids_h, tids_h, word_h, pos_h, typ_h, gam_h, bet_h,
                out_h, widx_v, pidx_v, tidx_v, buf_v, pos_v, typ_v,
                gam_v, bet_v, sem):
        wid = lax.axis_index("s") * info.num_cores + lax.axis_index("c")
        base_w = wid * rows_per_w

        pltpu.sync_copy(pos_h, pos_v)
        pltpu.sync_copy(typ_h, typ_v)
        pltpu.sync_copy(gam_h, gam_v)
        pltpu.sync_copy(bet_h, bet_v)

        def chunk_body(c, carry):
            base = base_w + c * chunk
            pltpu.sync_copy(ids_h.at[pl.ds(base, chunk)], widx_v)
            pltpu.sync_copy(pids_h.at[pl.ds(base, chunk)], pidx_v)
            pltpu.sync_copy(tids_h.at[pl.ds(base, chunk)], tidx_v)
            pltpu.async_copy(word_h.at[widx_v], buf_v, sem).wait()

            def group_body(gi, rcarry):
                pvec = pidx_v[pl.ds(gi * LANES, LANES)]
                tvec = tidx_v[pl.ds(gi * LANES, LANES)].astype(jnp.float32)
                for r16 in range(LANES):
                    r = gi * LANES + r16
                    pbase = pvec[r16] * H
                    tf = tvec[r16]
                    xs = []
                    for j in range(NJ):
                        w = buf_v[r, pl.ds(j * LANES, LANES)]
                        pv = pos_v[pl.ds(pbase + j * LANES, LANES)]
                        t0 = typ_v[pl.ds(j * LANES, LANES)]
                        t1 = typ_v[pl.ds(H + j * LANES, LANES)]
                        x = w + pv + (t0 + tf * (t1 - t0))
                        xs.append(x)
                    acc = xs[0]
                    acc2 = xs[0] * xs[0]
                    for j in range(1, NJ):
                        acc = acc + xs[j]
                        acc2 = acc2 + xs[j] * xs[j]
                    rs = jnp.sum(acc)
                    rq = jnp.sum(acc2)
                    mean = rs * (1.0 / H)
                    var = rq * (1.0 / H) - mean * mean
                    inv = _rsqrt_scalar(var + EPS)
                    for j in range(NJ):
                        g = gam_v[pl.ds(j * LANES, LANES)]
                        b = bet_v[pl.ds(j * LANES, LANES)]
                        buf_v[r, pl.ds(j * LANES, LANES)] = (
                            (xs[j] - mean) * inv * g + b)
                return rcarry

            lax.fori_loop(0, chunk // LANES, group_body, 0, unroll=False)
            pltpu.sync_copy(buf_v, out_h.at[pl.ds(base, chunk)])
            return carry

        lax.fori_loop(0, n_chunks, chunk_body, 0, unroll=False)

    return sc_call


def kernel(input_ids, position_ids, token_type_ids, word_embeddings,
           position_embeddings, token_type_embeddings, ln_gamma, ln_beta):
    b, l = input_ids.shape
    v, h = word_embeddings.shape
    p = position_embeddings.shape[0]
    n_rows = b * l
    ids = input_ids.reshape(n_rows).astype(jnp.int32)
    pids = position_ids.reshape(n_rows).astype(jnp.int32)
    tids = token_type_ids.reshape(n_rows).astype(jnp.int32)
    sc_call = _make_sc_call(n_rows, v, p)
    out = sc_call(ids, pids, tids, word_embeddings,
                  position_embeddings.reshape(p * h),
                  token_type_embeddings.reshape(2 * h),
                  ln_gamma, ln_beta)
    return out.reshape(b, l, h)


# speedup vs baseline: 8.3744x; 5.6540x over previous
import functools

import jax
import jax.numpy as jnp
from jax import lax
from jax.experimental import pallas as pl
from jax.experimental.pallas import tpu as pltpu
from jax.experimental.pallas import tpu_sc as plsc

H = 128
LANES = 16
NJ = H // LANES
EPS = 1e-12
CHUNK = 128


def _rsqrt_scalar(x):
    i = lax.bitcast_convert_type(x, jnp.int32)
    i = jnp.int32(0x5F3759DF) - (i >> 1)
    y = lax.bitcast_convert_type(i, jnp.float32)
    for _ in range(3):
        y = y * (1.5 - 0.5 * x * y * y)
    return y


def _make_sc_call(n_rows, v, pt_rows):
    info = plsc.get_sparse_core_info()
    nw = info.num_cores * info.num_subcores
    rows_per_w = n_rows // nw
    n_chunks = rows_per_w // CHUNK
    mesh = plsc.VectorSubcoreMesh(core_axis_name="c", subcore_axis_name="s")

    @functools.partial(
        pl.kernel,
        out_type=jax.ShapeDtypeStruct((n_rows, H), jnp.float32),
        mesh=mesh,
        scratch_types=[
            pltpu.VMEM((2, 2, CHUNK), jnp.int32),
            pltpu.VMEM((2, CHUNK, H), jnp.float32),
            pltpu.VMEM((2, CHUNK, H), jnp.float32),
            pltpu.VMEM((CHUNK, H), jnp.float32),
            pltpu.VMEM((H,), jnp.float32),
            pltpu.VMEM((H,), jnp.float32),
            pltpu.SemaphoreType.DMA,
            pltpu.SemaphoreType.DMA,
            pltpu.SemaphoreType.DMA,
        ],
        compiler_params=pltpu.CompilerParams(needs_layout_passes=False),
    )
    def sc_call(idx2_h, word_h, ptab_h, gam_h, bet_h, out_h,
                idx_v, wbuf_v, pbuf_v, obuf_v, gam_v, bet_v,
                gsem0, gsem1, osem):
        gsems = (gsem0, gsem1)
        wid = lax.axis_index("s") * info.num_cores + lax.axis_index("c")
        base_w = wid * rows_per_w

        pltpu.sync_copy(gam_h, gam_v)
        pltpu.sync_copy(bet_h, bet_v)
        gams = [gam_v[pl.ds(j * LANES, LANES)] for j in range(NJ)]
        bets = [bet_v[pl.ds(j * LANES, LANES)] for j in range(NJ)]

        def load_idx(c, s):
            pltpu.sync_copy(idx2_h.at[:, pl.ds(base_w + c * CHUNK, CHUNK)],
                            idx_v.at[s])

        def start_gathers(s):
            w = pltpu.async_copy(word_h.at[idx_v.at[s, 0]], wbuf_v.at[s],
                                 gsems[s])
            p = pltpu.async_copy(ptab_h.at[idx_v.at[s, 1]], pbuf_v.at[s],
                                 gsems[s])
            return w, p

        def wait_gathers(s):
            pltpu.make_async_copy(word_h.at[idx_v.at[s, 0]], wbuf_v.at[s],
                                  gsems[s]).wait()
            pltpu.make_async_copy(ptab_h.at[idx_v.at[s, 1]], pbuf_v.at[s],
                                  gsems[s]).wait()

        def out_handle(c):
            return pltpu.make_async_copy(
                obuf_v, out_h.at[pl.ds(base_w + c * CHUNK, CHUNK)], osem)

        load_idx(0, 0)
        start_gathers(0)
        load_idx(1, 1)
        start_gathers(1)

        def phase(s, c):
            wait_gathers(s)
            @pl.when(c > 0)
            def _():
                out_handle(c - 1).wait()

            def group_body(gi, rcarry):
                for r16 in range(LANES):
                    r = gi * LANES + r16
                    xs = []
                    for j in range(NJ):
                        w = wbuf_v[s, r, pl.ds(j * LANES, LANES)]
                        pv = pbuf_v[s, r, pl.ds(j * LANES, LANES)]
                        xs.append(w + pv)
                    acc = xs[0]
                    acc2 = xs[0] * xs[0]
                    for j in range(1, NJ):
                        acc = acc + xs[j]
                        acc2 = acc2 + xs[j] * xs[j]
                    rs = jnp.sum(acc)
                    rq = jnp.sum(acc2)
                    mean = rs * (1.0 / H)
                    var = rq * (1.0 / H) - mean * mean
                    inv = _rsqrt_scalar(var + EPS)
                    for j in range(NJ):
                        obuf_v[r, pl.ds(j * LANES, LANES)] = (
                            (xs[j] - mean) * inv * gams[j] + bets[j])
                return rcarry

            lax.fori_loop(0, CHUNK // LANES, group_body, 0, unroll=False)

            @pl.when(c + 2 < n_chunks)
            def _():
                load_idx(c + 2, s)
                start_gathers(s)

            out_handle(c).start()

        def pair_body(i, carry):
            phase(0, 2 * i)
            phase(1, 2 * i + 1)
            return carry

        lax.fori_loop(0, n_chunks // 2, pair_body, 0, unroll=False)
        out_handle(n_chunks - 1).wait()

    return sc_call


def kernel(input_ids, position_ids, token_type_ids, word_embeddings,
           position_embeddings, token_type_embeddings, ln_gamma, ln_beta):
    b, l = input_ids.shape
    v, h = word_embeddings.shape
    p = position_embeddings.shape[0]
    t = token_type_embeddings.shape[0]
    n_rows = b * l
    ids = input_ids.reshape(n_rows).astype(jnp.int32)
    ptids = (token_type_ids.reshape(n_rows).astype(jnp.int32) * p
             + position_ids.reshape(n_rows).astype(jnp.int32))
    idx2 = jnp.stack([ids, ptids])
    ptab = (position_embeddings[None, :, :]
            + token_type_embeddings[:, None, :]).reshape(t * p, h)
    sc_call = _make_sc_call(n_rows, v, t * p)
    out = sc_call(idx2, word_embeddings, ptab, ln_gamma, ln_beta)
    return out.reshape(b, l, h)


# speedup vs baseline: 8.6967x; 1.0385x over previous
import functools

import jax
import jax.numpy as jnp
from jax import lax
from jax.experimental import pallas as pl
from jax.experimental.pallas import tpu as pltpu
from jax.experimental.pallas import tpu_sc as plsc

H = 128
LANES = 16
NJ = H // LANES
EPS = 1e-12
CHUNK = 128


def _rsqrt_scalar(x):
    i = lax.bitcast_convert_type(x, jnp.int32)
    i = jnp.int32(0x5F3759DF) - (i >> 1)
    y = lax.bitcast_convert_type(i, jnp.float32)
    for _ in range(3):
        y = y * (1.5 - 0.5 * x * y * y)
    return y


def _make_sc_call(n_rows, v, pt_rows):
    info = plsc.get_sparse_core_info()
    nw = info.num_cores * info.num_subcores
    rows_per_w = n_rows // nw
    n_chunks = rows_per_w // CHUNK
    mesh = plsc.VectorSubcoreMesh(core_axis_name="c", subcore_axis_name="s")

    @functools.partial(
        pl.kernel,
        out_type=jax.ShapeDtypeStruct((n_rows, H), jnp.float32),
        mesh=mesh,
        scratch_types=[
            pltpu.VMEM((2, 2, CHUNK), jnp.int32),
            pltpu.VMEM((2, CHUNK, H), jnp.float32),
            pltpu.VMEM((CHUNK, H), jnp.float32),
            pltpu.VMEM((H,), jnp.float32),
            pltpu.VMEM((H,), jnp.float32),
            pltpu.SemaphoreType.DMA,
            pltpu.SemaphoreType.DMA,
            pltpu.SemaphoreType.DMA,
            pltpu.SemaphoreType.DMA,
            pltpu.SemaphoreType.DMA,
        ],
        compiler_params=pltpu.CompilerParams(needs_layout_passes=False),
    )
    def sc_call(idx2_h, word_h, ptab_h, gam_h, bet_h, out_h,
                idx_v, gbuf_v, obuf_v, gam_v, bet_v,
                wsem0, wsem1, psem0, psem1, osem):
        wsems = (wsem0, wsem1)
        psems = (psem0, psem1)
        wid = lax.axis_index("s") * info.num_cores + lax.axis_index("c")
        base_w = wid * rows_per_w

        pltpu.sync_copy(gam_h, gam_v)
        pltpu.sync_copy(bet_h, bet_v)

        def load_idx(c, s):
            pltpu.sync_copy(idx2_h.at[:, pl.ds(base_w + c * CHUNK, CHUNK)],
                            idx_v.at[s])

        def wgather(s):
            return pltpu.async_copy(word_h.at[idx_v.at[s, 0]], gbuf_v.at[s],
                                    wsems[s])

        def pgather(s):
            return pltpu.async_copy(ptab_h.at[idx_v.at[s, 1]], gbuf_v.at[s],
                                    psems[s], add=True)

        def wait_wgather(s):
            pltpu.make_async_copy(word_h.at[idx_v.at[s, 0]], gbuf_v.at[s],
                                  wsems[s]).wait()

        def wait_pgather(s):
            pltpu.make_async_copy(ptab_h.at[idx_v.at[s, 1]], gbuf_v.at[s],
                                  psems[s]).wait()

        def out_handle(c):
            return pltpu.make_async_copy(
                obuf_v, out_h.at[pl.ds(base_w + c * CHUNK, CHUNK)], osem)

        load_idx(0, 0)
        wgather(0)
        load_idx(1, 1)
        wgather(1)
        wait_wgather(0)
        pgather(0)

        def phase(s, c):
            o = 1 - s
            wait_pgather(s)

            @pl.when(c > 0)
            def _():
                out_handle(c - 1).wait()

            means = []
            invs = []

            def group_body(gi, rcarry):
                means.clear()
                invs.clear()
                for r16 in range(LANES):
                    r = gi * LANES + r16
                    x0 = gbuf_v[s, r, pl.ds(0, LANES)]
                    acc = x0
                    acc2 = x0 * x0
                    for j in range(1, NJ):
                        x = gbuf_v[s, r, pl.ds(j * LANES, LANES)]
                        acc = acc + x
                        acc2 = acc2 + x * x
                    rs = jnp.sum(acc)
                    rq = jnp.sum(acc2)
                    mean = rs * (1.0 / H)
                    var = rq * (1.0 / H) - mean * mean
                    means.append(mean)
                    invs.append(_rsqrt_scalar(var + EPS))
                for j in range(NJ):
                    g = gam_v[pl.ds(j * LANES, LANES)]
                    b = bet_v[pl.ds(j * LANES, LANES)]
                    for r16 in range(LANES):
                        r = gi * LANES + r16
                        x = gbuf_v[s, r, pl.ds(j * LANES, LANES)]
                        obuf_v[r, pl.ds(j * LANES, LANES)] = (
                            (x - means[r16]) * invs[r16]) * g + b
                return rcarry

            lax.fori_loop(0, CHUNK // LANES, group_body, 0, unroll=False)

            @pl.when(c + 2 < n_chunks)
            def _():
                load_idx(c + 2, s)
                wgather(s)

            @pl.when(c + 1 < n_chunks)
            def _():
                wait_wgather(o)
                pgather(o)

            out_handle(c).start()

        def pair_body(i, carry):
            phase(0, 2 * i)
            phase(1, 2 * i + 1)
            return carry

        lax.fori_loop(0, n_chunks // 2, pair_body, 0, unroll=False)
        out_handle(n_chunks - 1).wait()

    return sc_call


def kernel(input_ids, position_ids, token_type_ids, word_embeddings,
           position_embeddings, token_type_embeddings, ln_gamma, ln_beta):
    b, l = input_ids.shape
    v, h = word_embeddings.shape
    p = position_embeddings.shape[0]
    t = token_type_embeddings.shape[0]
    n_rows = b * l
    ids = input_ids.reshape(n_rows).astype(jnp.int32)
    ptids = (token_type_ids.reshape(n_rows).astype(jnp.int32) * p
             + position_ids.reshape(n_rows).astype(jnp.int32))
    idx2 = jnp.stack([ids, ptids])
    ptab = (position_embeddings[None, :, :]
            + token_type_embeddings[:, None, :]).reshape(t * p, h)
    sc_call = _make_sc_call(n_rows, v, t * p)
    out = sc_call(idx2, word_embeddings, ptab, ln_gamma, ln_beta)
    return out.reshape(b, l, h)
